# Initial kernel scaffold; baseline (speedup 1.0000x reference)
import jax
import jax.numpy as jnp
from jax.experimental import pallas as pl


def kernel(x, emb_word, emb2, emb3, W1, b1, W2, b2):
    raise NotImplementedError("write your pallas kernel here")


# speedup vs baseline: 4.1323x; 4.1323x over previous
import functools

import jax
import jax.numpy as jnp
from jax import lax
from jax.experimental import pallas as pl
from jax.experimental.pallas import tpu as pltpu
from jax.experimental.pallas import tpu_sc as plsc

_B = 4096
_S = 50
_E = 300
_EP = 304
_H = 256
_C = 4
_NC = 2
_NS = 16
_NW = _NC * _NS
_PER = _B // _NW
_NCHUNK = _E // 16
_TAIL = _E - 16


def _accum_rows(rows_ref):
    zero = jnp.zeros((16,), jnp.float32)

    def body(s, accs):
        new = [accs[j] + rows_ref[s, pl.ds(j * 16, 16)] for j in range(_NCHUNK)]
        new.append(accs[_NCHUNK] + rows_ref[s, pl.ds(_TAIL, 16)])
        return tuple(new)

    return lax.fori_loop(0, _S, body, tuple(zero for _ in range(_NCHUNK + 1)))


def _store_sums(st_ref, accs):
    st_ref[0, 0, pl.ds(288, 16)] = jnp.zeros((16,), jnp.float32)
    st_ref[0, 0, pl.ds(_TAIL, 16)] = accs[_NCHUNK]
    for j in range(_NCHUNK):
        st_ref[0, 0, pl.ds(j * 16, 16)] = accs[j]


def _sc_pooled_sums(xt, emb_word, emb2, emb3):
    mesh = plsc.VectorSubcoreMesh(core_axis_name="c", subcore_axis_name="s")

    @functools.partial(
        pl.kernel,
        out_type=jax.ShapeDtypeStruct((_B, 3, _EP), jnp.float32),
        mesh=mesh,
        scratch_types=[
            pltpu.VMEM((_PER, _S), jnp.int32),
            pltpu.VMEM((_S, _E), jnp.float32),
            pltpu.VMEM((_S, _E), jnp.float32),
            pltpu.VMEM((1, 1, _EP), jnp.float32),
            pltpu.VMEM((1, 1, _EP), jnp.float32),
            pltpu.SemaphoreType.DMA,
            pltpu.SemaphoreType.DMA,
            pltpu.SemaphoreType.DMA,
            pltpu.SemaphoreType.DMA,
        ],
    )
    def k(xt_hbm, w_hbm, g2_hbm, g3_hbm, out_hbm,
          idx_v, rows0, rows1, st0, st1, gs0, gs1, os0, os1):
        wid = lax.axis_index("s") * _NC + lax.axis_index("c")
        b0 = wid * _PER
        pltpu.sync_copy(xt_hbm.at[pl.ds(b0, _PER)], idx_v)

        rows = (rows0, rows1)
        sts = (st0, st1)
        gsems = (gs0, gs1)
        osems = (os0, os1)

        for t, tbl in enumerate((w_hbm, g2_hbm, g3_hbm)):
            def gstart(b, k_):
                def fire(i, s):
                    pltpu.make_async_copy(
                        tbl.at[pl.ds(i, 1)],
                        rows[k_].at[pl.ds(s, 1)],
                        gsems[k_]).start()
                for g in range(_S // 16):
                    vec = idx_v[b, pl.ds(g * 16, 16)]
                    for j in range(16):
                        fire(vec[j], g * 16 + j)
                vec = idx_v[b, pl.ds(_S - 16, 16)]
                for j in range(3 * 16 - (_S - 16), 16):
                    fire(vec[j], _S - 16 + j)

            def gwait(k_):
                def drain(s, _):
                    pltpu.make_async_copy(
                        tbl.at[pl.ds(0, 1)],
                        rows[k_].at[pl.ds(s, 1)],
                        gsems[k_]).wait()
                    return 0
                lax.fori_loop(0, _S, drain, 0)

            gstart(0, 0)
            gstart(1, 1)

            def step(b, k_):
                gwait(k_)
                accs = _accum_rows(rows[k_])

                @pl.when(b + 2 < _PER)
                def _():
                    gstart(b + 2, k_)

                out_cp = pltpu.make_async_copy(
                    sts[k_], out_hbm.at[pl.ds(b0 + b, 1), pl.ds(t, 1)],
                    osems[k_])
                if t == 0:
                    @pl.when(b >= 2)
                    def _():
                        pltpu.make_async_copy(
                            sts[k_],
                            out_hbm.at[pl.ds(b0 + b, 1), pl.ds(t, 1)],
                            osems[k_]).wait()
                else:
                    pltpu.make_async_copy(
                        sts[k_], out_hbm.at[pl.ds(b0 + b, 1), pl.ds(t, 1)],
                        osems[k_]).wait()
                _store_sums(sts[k_], accs)
                out_cp.start()

            def pair(b2, _):
                step(b2 * 2, 0)
                step(b2 * 2 + 1, 1)
                return 0

            lax.fori_loop(0, _PER // 2, pair, 0)

        pltpu.make_async_copy(
            st0, out_hbm.at[pl.ds(b0, 1), pl.ds(0, 1)], os0).wait()
        pltpu.make_async_copy(
            st1, out_hbm.at[pl.ds(b0, 1), pl.ds(1, 1)], os1).wait()

    return k(xt, emb_word, emb2, emb3)


def _tc_mlp(feats, w1p, b1, w2, b2):
    blk = 1024

    def body(s_ref, w1_ref, b1_ref, w2_ref, b2_ref, o_ref):
        h = jnp.dot(s_ref[...], w1_ref[...],
                    preferred_element_type=jnp.float32) + b1_ref[...]
        h = jnp.maximum(h, 0.0)
        logits = jnp.dot(h, w2_ref[...],
                         preferred_element_type=jnp.float32) + b2_ref[...]
        m = jnp.max(logits, axis=1, keepdims=True)
        e = jnp.exp(logits - m)
        lse = jnp.log(jnp.sum(e, axis=1, keepdims=True))
        o_ref[...] = logits - m - lse

    return pl.pallas_call(
        body,
        grid=(_B // blk,),
        in_specs=[
            pl.BlockSpec((blk, 3 * _EP), lambda i: (i, 0)),
            pl.BlockSpec((3 * _EP, _H), lambda i: (0, 0)),
            pl.BlockSpec((1, _H), lambda i: (0, 0)),
            pl.BlockSpec((_H, _C), lambda i: (0, 0)),
            pl.BlockSpec((1, _C), lambda i: (0, 0)),
        ],
        out_specs=pl.BlockSpec((blk, _C), lambda i: (i, 0)),
        out_shape=jax.ShapeDtypeStruct((_B, _C), jnp.float32),
    )(feats, w1p, b1.reshape(1, _H), w2, b2.reshape(1, _C))


def kernel(x, emb_word, emb2, emb3, W1, b1, W2, b2):
    xt = jnp.transpose(x)
    sums = _sc_pooled_sums(xt, emb_word, emb2, emb3)
    feats = sums.reshape(_B, 3 * _EP)
    w1p = jnp.zeros((3 * _EP, _H), W1.dtype)
    w1p = w1p.at[0:_E].set(W1[0:_E])
    w1p = w1p.at[_EP:_EP + _E].set(W1[_E:2 * _E])
    w1p = w1p.at[2 * _EP:2 * _EP + _E].set(W1[2 * _E:3 * _E])
    w1p = w1p * (1.0 / _S)
    return _tc_mlp(feats, w1p, b1, W2, b2)
